# baseline (speedup 1.0000x reference)
import functools

import jax
import jax.numpy as jnp
from jax import lax
from jax.experimental import pallas as pl
from jax.experimental.pallas import tpu as pltpu
from jax.experimental.pallas import tpu_sc as plsc

B, NC, NF, C, D = 4096, 26, 13, 1000, 128
NW = 32
RPW = B // NW
CB = 4
CHUNK = CB * NC
NSTEP = RPW // CB
NV = D // 16
NBUF = 6


RB = 512


def _dense_body(conts_ref, w_ref, bcat_ref, bcont_ref, out_ref):
    conts = conts_ref[...]
    mu = jnp.sum(conts, axis=0, keepdims=True) / B
    cz = conts - mu
    var = jnp.sum(cz * cz, axis=0, keepdims=True) / (B - 1)
    sd = jnp.sqrt(var)
    sd = jnp.where(sd > 0.0, sd, 1.0)
    z = cz / (sd + 1e-8)
    bias = (jnp.sum(bcat_ref[...], axis=0, keepdims=True)
            + jnp.sum(bcont_ref[...], axis=0, keepdims=True))
    out_ref[...] = (
        jnp.dot(z, w_ref[...], preferred_element_type=jnp.float32) + bias)


def _dense_part(conts, w_cont, b_cat, b_cont):
    return pl.pallas_call(
        _dense_body,
        out_shape=jax.ShapeDtypeStruct((B, D), jnp.float32),
    )(conts, w_cont, b_cat, b_cont)


def _add_body(a_ref, b_ref, out_ref):
    out_ref[...] = a_ref[...] + b_ref[...]


def _combine(dense, scsum):
    return pl.pallas_call(
        _add_body,
        grid=(B // RB,),
        in_specs=[pl.BlockSpec((RB, D), lambda i: (i, 0)),
                  pl.BlockSpec((RB, D), lambda i: (i, 0))],
        out_specs=pl.BlockSpec((RB, D), lambda i: (i, 0)),
        out_shape=jax.ShapeDtypeStruct((B, D), jnp.float32),
    )(dense, scsum)


def _sc_body(table_hbm, idx_hbm, out_hbm, idx_v,
             buf0, buf1, buf2, buf3, buf4, buf5,
             acc_v, sem0, sem1, sem2, sem3, sem4, sem5):
    cid = lax.axis_index("c")
    sid = lax.axis_index("s")
    wid = sid * 2 + cid
    base = wid * RPW

    pltpu.sync_copy(idx_hbm.at[pl.ds(wid * NSTEP, NSTEP)], idx_v)

    bufs = (buf0, buf1, buf2, buf3, buf4, buf5)
    sems = (sem0, sem1, sem2, sem3, sem4, sem5)
    cps = [None] * NBUF
    for j in range(NBUF - 1):
        cps[j] = pltpu.async_copy(table_hbm.at[idx_v.at[j]], bufs[j], sems[j])

    for j in range(NSTEP):
        p = j % NBUF
        if j + NBUF - 1 < NSTEP:
            q = (j + NBUF - 1) % NBUF
            cps[q] = pltpu.async_copy(
                table_hbm.at[idx_v.at[j + NBUF - 1]], bufs[q], sems[q])
        cps[p].wait()
        buf = bufs[p]

        def cb_body(cb, _, buf=buf, j=j):
            row = j * CB + cb

            def n_body(n, accs):
                r = cb * NC + n
                return tuple(accs[d] + buf[r, pl.ds(d * 16, 16)]
                             for d in range(NV))

            accs = tuple(buf[cb * NC, pl.ds(d * 16, 16)] for d in range(NV))
            accs = lax.fori_loop(1, NC, n_body, accs)
            for d in range(NV):
                acc_v[row, pl.ds(d * 16, 16)] = accs[d]
            return 0

        lax.fori_loop(0, CB, cb_body, 0)

    pltpu.sync_copy(acc_v, out_hbm.at[pl.ds(base, RPW)])


def _sc_gather_sum(table, idx2d):
    mesh = plsc.VectorSubcoreMesh(core_axis_name="c", subcore_axis_name="s",
                                  num_cores=2, num_subcores=16)
    f = pl.kernel(
        _sc_body, mesh=mesh,
        out_type=jax.ShapeDtypeStruct((B, D), jnp.float32),
        scratch_types=(
            [pltpu.VMEM((NSTEP, CHUNK), jnp.int32)]
            + [pltpu.VMEM((CHUNK, D), jnp.float32) for _ in range(NBUF)]
            + [pltpu.VMEM((RPW, D), jnp.float32)]
            + [pltpu.SemaphoreType.DMA for _ in range(NBUF)]
        ),
    )
    return f(table, idx2d)


def kernel(cats, conts, W_cat, b_cat, W_cont, b_cont):
    table = W_cat.reshape(NC * C, D)
    idx = (cats.astype(jnp.int32)
           + (jnp.arange(NC, dtype=jnp.int32) * C)[None, :])
    idx2d = idx.reshape(NW * NSTEP, CHUNK)
    scsum = _sc_gather_sum(table, idx2d)
    dense = _dense_part(conts, W_cont, b_cat, b_cont)
    return _combine(dense, scsum)


# speedup vs baseline: 1.0028x; 1.0028x over previous
import functools

import jax
import jax.numpy as jnp
from jax import lax
from jax.experimental import pallas as pl
from jax.experimental.pallas import tpu as pltpu
from jax.experimental.pallas import tpu_sc as plsc

B, NC, NF, C, D = 4096, 26, 13, 1000, 128
NW = 32
RPW = B // NW
CB = 4
CHUNK = CB * NC
NSTEP = RPW // CB
NV = D // 16
NBUF = 6


RB = 512


def _dense_body(conts_ref, w_ref, bcat_ref, bcont_ref, out_ref):
    conts = conts_ref[...]
    mu = jnp.sum(conts, axis=0, keepdims=True) / B
    cz = conts - mu
    var = jnp.sum(cz * cz, axis=0, keepdims=True) / (B - 1)
    sd = jnp.sqrt(var)
    sd = jnp.where(sd > 0.0, sd, 1.0)
    z = cz / (sd + 1e-8)
    bias = (jnp.sum(bcat_ref[...], axis=0, keepdims=True)
            + jnp.sum(bcont_ref[...], axis=0, keepdims=True))
    out_ref[...] = (
        jnp.dot(z, w_ref[...], preferred_element_type=jnp.float32) + bias)


def _dense_part(conts, w_cont, b_cat, b_cont):
    return pl.pallas_call(
        _dense_body,
        out_shape=jax.ShapeDtypeStruct((B, D), jnp.float32),
    )(conts, w_cont, b_cat, b_cont)


def _sc_body(table_hbm, idx_hbm, dense_hbm, out_hbm, idx_v,
             buf0, buf1, buf2, buf3, buf4, buf5,
             acc_v, sem0, sem1, sem2, sem3, sem4, sem5):
    cid = lax.axis_index("c")
    sid = lax.axis_index("s")
    wid = sid * 2 + cid
    base = wid * RPW

    pltpu.sync_copy(idx_hbm.at[pl.ds(wid * NSTEP, NSTEP)], idx_v)
    pltpu.sync_copy(dense_hbm.at[pl.ds(base, RPW)], acc_v)

    bufs = (buf0, buf1, buf2, buf3, buf4, buf5)
    sems = (sem0, sem1, sem2, sem3, sem4, sem5)
    cps = [None] * NBUF
    for j in range(NBUF - 1):
        cps[j] = pltpu.async_copy(table_hbm.at[idx_v.at[j]], bufs[j], sems[j])

    for j in range(NSTEP):
        p = j % NBUF
        if j + NBUF - 1 < NSTEP:
            q = (j + NBUF - 1) % NBUF
            cps[q] = pltpu.async_copy(
                table_hbm.at[idx_v.at[j + NBUF - 1]], bufs[q], sems[q])
        cps[p].wait()
        buf = bufs[p]

        def cb_body(cb, _, buf=buf, j=j):
            row = j * CB + cb

            def n_body(n, accs):
                r = cb * NC + n
                return tuple(accs[d] + buf[r, pl.ds(d * 16, 16)]
                             for d in range(NV))

            accs = tuple(acc_v[row, pl.ds(d * 16, 16)] for d in range(NV))
            accs = lax.fori_loop(0, NC, n_body, accs)
            for d in range(NV):
                acc_v[row, pl.ds(d * 16, 16)] = accs[d]
            return 0

        lax.fori_loop(0, CB, cb_body, 0)

    pltpu.sync_copy(acc_v, out_hbm.at[pl.ds(base, RPW)])


def _sc_gather_sum(table, idx2d, dense):
    mesh = plsc.VectorSubcoreMesh(core_axis_name="c", subcore_axis_name="s",
                                  num_cores=2, num_subcores=16)
    f = pl.kernel(
        _sc_body, mesh=mesh,
        out_type=jax.ShapeDtypeStruct((B, D), jnp.float32),
        scratch_types=(
            [pltpu.VMEM((NSTEP, CHUNK), jnp.int32)]
            + [pltpu.VMEM((CHUNK, D), jnp.float32) for _ in range(NBUF)]
            + [pltpu.VMEM((RPW, D), jnp.float32)]
            + [pltpu.SemaphoreType.DMA for _ in range(NBUF)]
        ),
    )
    return f(table, idx2d, dense)


def kernel(cats, conts, W_cat, b_cat, W_cont, b_cont):
    table = W_cat.reshape(NC * C, D)
    idx = (cats.astype(jnp.int32)
           + (jnp.arange(NC, dtype=jnp.int32) * C)[None, :])
    idx2d = idx.reshape(NW * NSTEP, CHUNK)
    dense = _dense_part(conts, W_cont, b_cat, b_cont)
    return _sc_gather_sum(table, idx2d, dense)


# speedup vs baseline: 1.0272x; 1.0244x over previous
import functools

import jax
import jax.numpy as jnp
from jax import lax
from jax.experimental import pallas as pl
from jax.experimental.pallas import tpu as pltpu
from jax.experimental.pallas import tpu_sc as plsc

B, NC, NF, C, D = 4096, 26, 13, 1000, 128
NW = 32
RPW = B // NW
CB = 4
CHUNK = CB * NC
NSTEP = RPW // CB
NV = D // 16
NBUF = 6


def _combine_body(conts_ref, w_ref, bcat_ref, bcont_ref, scsum_ref, out_ref):
    conts = conts_ref[...]
    mu = jnp.sum(conts, axis=0, keepdims=True) / B
    cz = conts - mu
    var = jnp.sum(cz * cz, axis=0, keepdims=True) / (B - 1)
    sd = jnp.sqrt(var)
    sd = jnp.where(sd > 0.0, sd, 1.0)
    z = cz / (sd + 1e-8)
    bias = (jnp.sum(bcat_ref[...], axis=0, keepdims=True)
            + jnp.sum(bcont_ref[...], axis=0, keepdims=True))
    out_ref[...] = (
        scsum_ref[...]
        + jnp.dot(z, w_ref[...], preferred_element_type=jnp.float32) + bias)


def _combine(conts, w_cont, b_cat, b_cont, scsum):
    return pl.pallas_call(
        _combine_body,
        out_shape=jax.ShapeDtypeStruct((B, D), jnp.float32),
    )(conts, w_cont, b_cat, b_cont, scsum)


def _sc_body(table_hbm, idx_hbm, out_hbm, idx_v,
             buf0, buf1, buf2, buf3, buf4, buf5,
             acc_v, sem0, sem1, sem2, sem3, sem4, sem5):
    cid = lax.axis_index("c")
    sid = lax.axis_index("s")
    wid = sid * 2 + cid
    base = wid * RPW

    pltpu.sync_copy(idx_hbm.at[pl.ds(wid * (NSTEP * CHUNK), NSTEP * CHUNK)],
                    idx_v)

    bufs = (buf0, buf1, buf2, buf3, buf4, buf5)
    sems = (sem0, sem1, sem2, sem3, sem4, sem5)
    cps = [None] * NBUF
    for j in range(NBUF - 1):
        cps[j] = pltpu.async_copy(
            table_hbm.at[idx_v.at[pl.ds(j * CHUNK, CHUNK)]], bufs[j], sems[j])

    for j in range(NSTEP):
        p = j % NBUF
        if j + NBUF - 1 < NSTEP:
            q = (j + NBUF - 1) % NBUF
            cps[q] = pltpu.async_copy(
                table_hbm.at[idx_v.at[pl.ds((j + NBUF - 1) * CHUNK, CHUNK)]],
                bufs[q], sems[q])
        cps[p].wait()
        buf = bufs[p]

        def cb_body(cb, _, buf=buf, j=j):
            row = j * CB + cb

            def n_body(n, accs):
                r = cb * NC + n
                return tuple(accs[d] + buf[r, pl.ds(d * 16, 16)]
                             for d in range(NV))

            accs = tuple(buf[cb * NC, pl.ds(d * 16, 16)] for d in range(NV))
            accs = lax.fori_loop(1, NC, n_body, accs)
            for d in range(NV):
                acc_v[row, pl.ds(d * 16, 16)] = accs[d]
            return 0

        lax.fori_loop(0, CB, cb_body, 0)

    pltpu.sync_copy(acc_v, out_hbm.at[pl.ds(base, RPW)])


def _sc_gather_sum(table, idx2d):
    mesh = plsc.VectorSubcoreMesh(core_axis_name="c", subcore_axis_name="s",
                                  num_cores=2, num_subcores=16)
    f = pl.kernel(
        _sc_body, mesh=mesh,
        out_type=jax.ShapeDtypeStruct((B, D), jnp.float32),
        scratch_types=(
            [pltpu.VMEM((NSTEP * CHUNK,), jnp.int32)]
            + [pltpu.VMEM((CHUNK, D), jnp.float32) for _ in range(NBUF)]
            + [pltpu.VMEM((RPW, D), jnp.float32)]
            + [pltpu.SemaphoreType.DMA for _ in range(NBUF)]
        ),
    )
    return f(table, idx2d)


def kernel(cats, conts, W_cat, b_cat, W_cont, b_cont):
    table = W_cat.reshape(NC * C, D)
    idx = (cats.astype(jnp.int32)
           + (jnp.arange(NC, dtype=jnp.int32) * C)[None, :])
    idx1d = idx.reshape(NW * NSTEP * CHUNK)
    scsum = _sc_gather_sum(table, idx1d)
    return _combine(conts, W_cont, b_cat, b_cont, scsum)


# speedup vs baseline: 1.0432x; 1.0155x over previous
import functools

import jax
import jax.numpy as jnp
from jax import lax
from jax.experimental import pallas as pl
from jax.experimental.pallas import tpu as pltpu
from jax.experimental.pallas import tpu_sc as plsc

B, NC, NF, C, D = 4096, 26, 13, 1000, 128
NW = 32
RPW = B // NW
CB = 4
CHUNK = CB * NC
NSTEP = RPW // CB
NV = D // 16
NBUF = 6


def _combine_body(conts_ref, w_ref, bcat_ref, bcont_ref, scsum_ref, out_ref):
    conts = conts_ref[...]
    mu = jnp.sum(conts, axis=0, keepdims=True) / B
    cz = conts - mu
    var = jnp.sum(cz * cz, axis=0, keepdims=True) / (B - 1)
    sd = jnp.sqrt(var)
    sd = jnp.where(sd > 0.0, sd, 1.0)
    z = cz / (sd + 1e-8)
    bias = (jnp.sum(bcat_ref[...], axis=0, keepdims=True)
            + jnp.sum(bcont_ref[...], axis=0, keepdims=True))
    out_ref[...] = (
        scsum_ref[...]
        + jnp.dot(z, w_ref[...], preferred_element_type=jnp.float32) + bias)


def _combine(conts, w_cont, b_cat, b_cont, scsum):
    return pl.pallas_call(
        _combine_body,
        out_shape=jax.ShapeDtypeStruct((B, D), jnp.float32),
    )(conts, w_cont, b_cat, b_cont, scsum)


def _sc_body(table_hbm, idx_hbm, out_hbm, idx_v,
             buf0, buf1, buf2, buf3, buf4, buf5,
             acc_v, sem0, sem1, sem2, sem3, sem4, sem5):
    cid = lax.axis_index("c")
    sid = lax.axis_index("s")
    wid = sid * 2 + cid
    base = wid * RPW

    pltpu.sync_copy(idx_hbm.at[pl.ds(wid * NSTEP, NSTEP)], idx_v)

    bufs = (buf0, buf1, buf2, buf3, buf4, buf5)
    sems = (sem0, sem1, sem2, sem3, sem4, sem5)
    cps = [None] * NBUF
    for j in range(NBUF - 1):
        cps[j] = pltpu.async_copy(table_hbm.at[idx_v.at[j]], bufs[j], sems[j])

    for j in range(NSTEP):
        p = j % NBUF
        if j + NBUF - 1 < NSTEP:
            q = (j + NBUF - 1) % NBUF
            cps[q] = pltpu.async_copy(
                table_hbm.at[idx_v.at[j + NBUF - 1]], bufs[q], sems[q])
        cps[p].wait()
        buf = bufs[p]

        def cb_body(cb, _, buf=buf, j=j):
            row = j * CB + cb

            def n_body(n, accs):
                r = cb * NC + n
                return tuple(accs[d] + buf[r, pl.ds(d * 16, 16)]
                             for d in range(NV))

            accs = tuple(buf[cb * NC, pl.ds(d * 16, 16)] for d in range(NV))
            accs = lax.fori_loop(1, NC, n_body, accs)
            for d in range(NV):
                acc_v[row, pl.ds(d * 16, 16)] = accs[d]
            return 0

        lax.fori_loop(0, CB, cb_body, 0)

    pltpu.sync_copy(acc_v, out_hbm.at[pl.ds(base, RPW)])


def _sc_gather_sum(table, idx2d):
    mesh = plsc.VectorSubcoreMesh(core_axis_name="c", subcore_axis_name="s",
                                  num_cores=2, num_subcores=16)
    f = pl.kernel(
        _sc_body, mesh=mesh,
        out_type=jax.ShapeDtypeStruct((B, D), jnp.float32),
        scratch_types=(
            [pltpu.VMEM((NSTEP, CHUNK), jnp.int32)]
            + [pltpu.VMEM((CHUNK, D), jnp.float32) for _ in range(NBUF)]
            + [pltpu.VMEM((RPW, D), jnp.float32)]
            + [pltpu.SemaphoreType.DMA for _ in range(NBUF)]
        ),
    )
    return f(table, idx2d)


def kernel(cats, conts, W_cat, b_cat, W_cont, b_cont):
    table = W_cat.reshape(NC * C, D)
    idx = (cats.astype(jnp.int32)
           + (jnp.arange(NC, dtype=jnp.int32) * C)[None, :])
    idx2d = idx.reshape(NW * NSTEP, CHUNK)
    scsum = _sc_gather_sum(table, idx2d)
    return _combine(conts, W_cont, b_cat, b_cont, scsum)
